# Initial kernel scaffold; baseline (speedup 1.0000x reference)
import jax
import jax.numpy as jnp
from jax.experimental import pallas as pl


def kernel(x, table, W, b, pos_embed):
    raise NotImplementedError("write your pallas kernel here")


# speedup vs baseline: 17.6511x; 17.6511x over previous
import functools

import jax
import jax.numpy as jnp
from jax import lax
from jax.experimental import pallas as pl
from jax.experimental.pallas import tpu as pltpu
from jax.experimental.pallas import tpu_sc as plsc

_NC = 2
_NS = 16
_NW = _NC * _NS
_IDXW = 128


def _sc_gather_mean(x2, table, *, n_patches_total, patch, cp):
    pw = n_patches_total // _NW
    n_chunks = pw // cp
    ct = cp * patch
    n_idx_rows = ct // _IDXW

    def body(x_hbm, table_hbm, feat_hbm, idx_v, rows_v, out_v, sem):
        wid = lax.axis_index("s") * _NC + lax.axis_index("c")

        def chunk(c, carry):
            xrow0 = wid * (n_chunks * n_idx_rows) + c * n_idx_rows
            pltpu.sync_copy(x_hbm.at[pl.ds(xrow0, n_idx_rows)], idx_v)
            cps = []
            for j in range(n_idx_rows):
                cps.append(pltpu.async_copy(
                    table_hbm.at[idx_v.at[j]],
                    rows_v.at[pl.ds(j * _IDXW, _IDXW)],
                    sem))
            for c_ in cps:
                c_.wait()

            def red(p, carry2):
                r0 = p * patch
                acc = rows_v[r0]
                for t in range(1, patch):
                    acc = acc + rows_v[r0 + t]
                out_v[p] = acc * (1.0 / patch)
                return carry2

            lax.fori_loop(0, cp, red, 0, unroll=False)
            pbase = wid * pw + c * cp
            pltpu.sync_copy(out_v, feat_hbm.at[pl.ds(pbase, cp)])
            return carry

        lax.fori_loop(0, n_chunks, chunk, 0, unroll=False)

    k = pl.kernel(
        body,
        out_type=jax.ShapeDtypeStruct((n_patches_total, patch), jnp.float32),
        mesh=plsc.VectorSubcoreMesh(core_axis_name="c", subcore_axis_name="s"),
        scratch_types=[
            pltpu.VMEM((n_idx_rows, _IDXW), jnp.int32),
            pltpu.VMEM((ct, patch), jnp.float32),
            pltpu.VMEM((cp, patch), jnp.float32),
            pltpu.SemaphoreType.DMA,
        ],
        compiler_params=pltpu.CompilerParams(use_tc_tiling_on_sc=False),
    )
    return k(x2, table)


def _tc_body(feat_ref, w_ref, b_ref, pos_ref, out_ref):
    f = feat_ref[...]
    bb, pp, k = f.shape
    acc = lax.dot_general(
        f.reshape(bb * pp, k), w_ref[...],
        (((1,), (0,)), ((), ())), preferred_element_type=jnp.float32)
    acc = acc.reshape(bb, pp, -1)
    out_ref[...] = acc + (pos_ref[...] + b_ref[...])[None, :, :]


def _tc_project(feat3, W, b, pos2, *, bb):
    B_, P_, patch = feat3.shape
    D_ = W.shape[1]
    return pl.pallas_call(
        _tc_body,
        grid=(B_ // bb,),
        in_specs=[
            pl.BlockSpec((bb, P_, patch), lambda i: (i, 0, 0)),
            pl.BlockSpec((patch, D_), lambda i: (0, 0)),
            pl.BlockSpec((D_,), lambda i: (0,)),
            pl.BlockSpec((P_, D_), lambda i: (0, 0)),
        ],
        out_specs=pl.BlockSpec((bb, P_, D_), lambda i: (i, 0, 0)),
        out_shape=jax.ShapeDtypeStruct((B_, P_, D_), jnp.float32),
        compiler_params=pltpu.CompilerParams(
            dimension_semantics=("parallel",)),
    )(feat3, W, b, pos2)


def kernel(x, table, W, b, pos_embed):
    B_, S_ = x.shape
    V_, patch = table.shape
    D_ = W.shape[1]
    n_patches = S_ // patch
    n_patches_total = B_ * n_patches

    x2 = x.reshape(n_patches_total * patch // _IDXW, _IDXW)
    feat = _sc_gather_mean(x2, table, n_patches_total=n_patches_total,
                           patch=patch, cp=128)
    feat3 = feat.reshape(B_, n_patches, patch)
    pos2 = pos_embed[0, :n_patches, :]
    return _tc_project(feat3, W, b, pos2, bb=8)


# speedup vs baseline: 22.6264x; 1.2819x over previous
import functools

import jax
import jax.numpy as jnp
from jax import lax
from jax.experimental import pallas as pl
from jax.experimental.pallas import tpu as pltpu
from jax.experimental.pallas import tpu_sc as plsc

_NC = 2
_NS = 16
_NW = _NC * _NS
_IDXW = 128


def _sc_gather_mean(x2, table, *, n_patches_total, patch, cp):
    pw = n_patches_total // _NW
    n_chunks = pw // cp
    ct = cp * patch
    n_idx_rows = ct // _IDXW
    assert n_chunks >= 4

    def body(x_hbm, table_hbm, feat_hbm, idx_v0, idx_v1, rows_v0, rows_v1,
             out_v0, out_v1, sem_g0, sem_g1, sem_o0, sem_o1):
        idx_v = (idx_v0, idx_v1)
        rows_v = (rows_v0, rows_v1)
        out_v = (out_v0, out_v1)
        sem_g = (sem_g0, sem_g1)
        sem_o = (sem_o0, sem_o1)
        wid = lax.axis_index("s") * _NC + lax.axis_index("c")

        def fire(c, b):
            xrow0 = wid * (n_chunks * n_idx_rows) + c * n_idx_rows
            pltpu.sync_copy(x_hbm.at[pl.ds(xrow0, n_idx_rows)], idx_v[b])
            for j in range(n_idx_rows):
                pltpu.async_copy(
                    table_hbm.at[idx_v[b].at[j]],
                    rows_v[b].at[pl.ds(j * _IDXW, _IDXW)],
                    sem_g[b])

        def wait_gathers(b):
            for j in range(n_idx_rows):
                pltpu.make_async_copy(
                    table_hbm.at[idx_v[b].at[j]],
                    rows_v[b].at[pl.ds(j * _IDXW, _IDXW)],
                    sem_g[b]).wait()

        def reduce_and_store(c, b, *, wait_out):
            if wait_out:
                pltpu.make_async_copy(
                    out_v[b], feat_hbm.at[pl.ds(0, cp)], sem_o[b]).wait()

            def red(p, carry2):
                r0 = p * patch
                acc = rows_v[b][r0]
                for t in range(1, patch):
                    acc = acc + rows_v[b][r0 + t]
                out_v[b][p] = acc * (1.0 / patch)
                return carry2

            lax.fori_loop(0, cp, red, 0, unroll=2)
            pbase = wid * pw + c * cp
            pltpu.async_copy(out_v[b], feat_hbm.at[pl.ds(pbase, cp)],
                             sem_o[b])

        fire(0, 0)
        fire(1, 1)
        for c in (0, 1):
            b = c & 1
            wait_gathers(b)
            reduce_and_store(c, b, wait_out=False)
            fire(c + 2, b)

        def main(i, carry):
            for b in range(2):
                c = 2 + 2 * i + b
                wait_gathers(b)
                reduce_and_store(c, b, wait_out=True)
                fire(c + 2, b)
            return carry

        assert n_chunks % 2 == 0
        if n_chunks > 4:
            lax.fori_loop(0, (n_chunks - 4) // 2, main, 0, unroll=False)

        for c in (n_chunks - 2, n_chunks - 1):
            b = c & 1
            wait_gathers(b)
            reduce_and_store(c, b, wait_out=True)
        for b in range(2):
            pltpu.make_async_copy(
                out_v[b], feat_hbm.at[pl.ds(0, cp)], sem_o[b]).wait()

    k = pl.kernel(
        body,
        out_type=jax.ShapeDtypeStruct((n_patches_total, patch), jnp.float32),
        mesh=plsc.VectorSubcoreMesh(core_axis_name="c", subcore_axis_name="s"),
        scratch_types=[
            pltpu.VMEM((n_idx_rows, _IDXW), jnp.int32),
            pltpu.VMEM((n_idx_rows, _IDXW), jnp.int32),
            pltpu.VMEM((ct, patch), jnp.float32),
            pltpu.VMEM((ct, patch), jnp.float32),
            pltpu.VMEM((cp, patch), jnp.float32),
            pltpu.VMEM((cp, patch), jnp.float32),
            pltpu.SemaphoreType.DMA,
            pltpu.SemaphoreType.DMA,
            pltpu.SemaphoreType.DMA,
            pltpu.SemaphoreType.DMA,
        ],
        compiler_params=pltpu.CompilerParams(use_tc_tiling_on_sc=False),
    )
    return k(x2, table)


def _tc_body(feat_ref, w_ref, b_ref, pos_ref, out_ref):
    f = feat_ref[...]
    bb, pp, k = f.shape
    acc = lax.dot_general(
        f.reshape(bb * pp, k), w_ref[...],
        (((1,), (0,)), ((), ())), preferred_element_type=jnp.float32)
    acc = acc.reshape(bb, pp, -1)
    out_ref[...] = acc + (pos_ref[...] + b_ref[...])[None, :, :]


def _tc_project(feat3, W, b, pos2, *, bb):
    B_, P_, patch = feat3.shape
    D_ = W.shape[1]
    return pl.pallas_call(
        _tc_body,
        grid=(B_ // bb,),
        in_specs=[
            pl.BlockSpec((bb, P_, patch), lambda i: (i, 0, 0)),
            pl.BlockSpec((patch, D_), lambda i: (0, 0)),
            pl.BlockSpec((D_,), lambda i: (0,)),
            pl.BlockSpec((P_, D_), lambda i: (0, 0)),
        ],
        out_specs=pl.BlockSpec((bb, P_, D_), lambda i: (i, 0, 0)),
        out_shape=jax.ShapeDtypeStruct((B_, P_, D_), jnp.float32),
        compiler_params=pltpu.CompilerParams(
            dimension_semantics=("parallel",)),
    )(feat3, W, b, pos2)


def kernel(x, table, W, b, pos_embed):
    B_, S_ = x.shape
    V_, patch = table.shape
    D_ = W.shape[1]
    n_patches = S_ // patch
    n_patches_total = B_ * n_patches

    x2 = x.reshape(n_patches_total * patch // _IDXW, _IDXW)
    feat = _sc_gather_mean(x2, table, n_patches_total=n_patches_total,
                           patch=patch, cp=128)
    feat3 = feat.reshape(B_, n_patches, patch)
    pos2 = pos_embed[0, :n_patches, :]
    return _tc_project(feat3, W, b, pos2, bb=8)


# speedup vs baseline: 23.1981x; 1.0253x over previous
import functools

import jax
import jax.numpy as jnp
from jax import lax
from jax.experimental import pallas as pl
from jax.experimental.pallas import tpu as pltpu
from jax.experimental.pallas import tpu_sc as plsc

_NC = 2
_NS = 16
_NW = _NC * _NS
_IDXW = 128


def _sc_gather_mean(x2, table, *, n_patches_total, patch, cp):
    pw = n_patches_total // _NW
    n_chunks = pw // cp
    ct = cp * patch
    n_idx_rows = ct // _IDXW
    assert pw % cp == 0 and ct % _IDXW == 0
    assert n_chunks >= 4 and n_chunks % 2 == 0

    def body(x_hbm, table_hbm, feat_hbm, idx_v0, idx_v1, rows_v0, rows_v1,
             out_v0, out_v1, sem_i0, sem_i1, sem_g0, sem_g1, sem_o0, sem_o1):
        idx_v = (idx_v0, idx_v1)
        rows_v = (rows_v0, rows_v1)
        out_v = (out_v0, out_v1)
        sem_i = (sem_i0, sem_i1)
        sem_g = (sem_g0, sem_g1)
        sem_o = (sem_o0, sem_o1)
        wid = lax.axis_index("s") * _NC + lax.axis_index("c")

        def stage_idx(c, b):
            xrow0 = wid * (n_chunks * n_idx_rows) + c * n_idx_rows
            pltpu.async_copy(x_hbm.at[pl.ds(xrow0, n_idx_rows)], idx_v[b],
                             sem_i[b])

        def fire_gathers(c, b):
            pltpu.make_async_copy(
                x_hbm.at[pl.ds(0, n_idx_rows)], idx_v[b], sem_i[b]).wait()
            for j in range(n_idx_rows):
                pltpu.async_copy(
                    table_hbm.at[idx_v[b].at[j]],
                    rows_v[b].at[pl.ds(j * _IDXW, _IDXW)],
                    sem_g[b])

        def wait_gathers(b):
            for j in range(n_idx_rows):
                pltpu.make_async_copy(
                    table_hbm.at[idx_v[b].at[j]],
                    rows_v[b].at[pl.ds(j * _IDXW, _IDXW)],
                    sem_g[b]).wait()

        def reduce_and_store(c, b, *, wait_out):
            if wait_out:
                pltpu.make_async_copy(
                    out_v[b], feat_hbm.at[pl.ds(0, cp)], sem_o[b]).wait()

            def red(p, carry2):
                r0 = p * patch
                acc = rows_v[b][r0]
                for t in range(1, patch):
                    acc = acc + rows_v[b][r0 + t]
                out_v[b][p] = acc * (1.0 / patch)
                return carry2

            lax.fori_loop(0, cp, red, 0, unroll=4)
            pbase = wid * pw + c * cp
            pltpu.async_copy(out_v[b], feat_hbm.at[pl.ds(pbase, cp)],
                             sem_o[b])

        stage_idx(0, 0)
        stage_idx(1, 1)
        fire_gathers(0, 0)
        fire_gathers(1, 1)
        for c in (0, 1):
            b = c & 1
            wait_gathers(b)
            stage_idx(c + 2, b)
            reduce_and_store(c, b, wait_out=False)
            fire_gathers(c + 2, b)

        def main(i, carry):
            for b in range(2):
                c = 2 + 2 * i + b
                wait_gathers(b)
                stage_idx(c + 2, b)
                reduce_and_store(c, b, wait_out=True)
                fire_gathers(c + 2, b)
            return carry

        if n_chunks > 4:
            lax.fori_loop(0, (n_chunks - 4) // 2, main, 0, unroll=False)

        for c in (n_chunks - 2, n_chunks - 1):
            b = c & 1
            wait_gathers(b)
            reduce_and_store(c, b, wait_out=True)
        for b in range(2):
            pltpu.make_async_copy(
                out_v[b], feat_hbm.at[pl.ds(0, cp)], sem_o[b]).wait()

    k = pl.kernel(
        body,
        out_type=jax.ShapeDtypeStruct((n_patches_total, patch), jnp.float32),
        mesh=plsc.VectorSubcoreMesh(core_axis_name="c", subcore_axis_name="s"),
        scratch_types=[
            pltpu.VMEM((n_idx_rows, _IDXW), jnp.int32),
            pltpu.VMEM((n_idx_rows, _IDXW), jnp.int32),
            pltpu.VMEM((ct, patch), jnp.float32),
            pltpu.VMEM((ct, patch), jnp.float32),
            pltpu.VMEM((cp, patch), jnp.float32),
            pltpu.VMEM((cp, patch), jnp.float32),
            pltpu.SemaphoreType.DMA,
            pltpu.SemaphoreType.DMA,
            pltpu.SemaphoreType.DMA,
            pltpu.SemaphoreType.DMA,
            pltpu.SemaphoreType.DMA,
            pltpu.SemaphoreType.DMA,
        ],
        compiler_params=pltpu.CompilerParams(use_tc_tiling_on_sc=False),
    )
    return k(x2, table)


def _tc_body(buf_ref, feat_ref, w_ref, b_ref, pos_ref, out_ref):
    f = feat_ref[...]
    bb, pp, k = f.shape
    acc = lax.dot_general(
        f.reshape(bb * pp, k), w_ref[...],
        (((1,), (0,)), ((), ())), preferred_element_type=jnp.float32)
    acc = acc.reshape(bb, pp, -1)
    out_ref[...] = acc + (pos_ref[...] + b_ref[...])[None, :, :]


def _tc_body_noalias(feat_ref, w_ref, b_ref, pos_ref, out_ref):
    _tc_body(None, feat_ref, w_ref, b_ref, pos_ref, out_ref)


def _tc_project_slice(buf, feat3_s, W, b, pos2, *, s, b_total, bb):
    bs, P_, patch = feat3_s.shape
    D_ = W.shape[1]
    nb = bs // bb
    specs = [
        pl.BlockSpec((bb, P_, patch), lambda i: (i, 0, 0)),
        pl.BlockSpec((patch, D_), lambda i: (0, 0)),
        pl.BlockSpec((D_,), lambda i: (0,)),
        pl.BlockSpec((P_, D_), lambda i: (0, 0)),
    ]
    out_spec = pl.BlockSpec((bb, P_, D_), lambda i, s=s: (s * nb + i, 0, 0))
    out_shape = jax.ShapeDtypeStruct((b_total, P_, D_), jnp.float32)
    params = pltpu.CompilerParams(dimension_semantics=("arbitrary",))
    if buf is None:
        return pl.pallas_call(
            _tc_body_noalias, grid=(nb,), in_specs=specs,
            out_specs=out_spec, out_shape=out_shape,
            compiler_params=params,
        )(feat3_s, W, b, pos2)
    return pl.pallas_call(
        _tc_body, grid=(nb,),
        in_specs=[pl.BlockSpec((1, 8, 128), lambda i: (0, 0, 0))] + specs,
        out_specs=out_spec, out_shape=out_shape,
        input_output_aliases={0: 0},
        compiler_params=params,
    )(buf, feat3_s, W, b, pos2)


def kernel(x, table, W, b, pos_embed):
    B_, S_ = x.shape
    V_, patch = table.shape
    D_ = W.shape[1]
    n_patches = S_ // patch
    pos2 = pos_embed[0, :n_patches, :]

    nsplit = 4
    bs = B_ // nsplit
    npt_s = bs * n_patches
    feats = []
    for s in range(nsplit):
        x_s = x[s * bs:(s + 1) * bs]
        x2 = x_s.reshape(npt_s * patch // _IDXW, _IDXW)
        feats.append(_sc_gather_mean(x2, table, n_patches_total=npt_s,
                                     patch=patch, cp=160))
    buf = None
    for s in range(nsplit):
        feat3_s = feats[s].reshape(bs, n_patches, patch)
        buf = _tc_project_slice(buf, feat3_s, W, b, pos2,
                                s=s, b_total=B_, bb=8)
    return buf


# speedup vs baseline: 23.2934x; 1.0041x over previous
import functools

import jax
import jax.numpy as jnp
from jax import lax
from jax.experimental import pallas as pl
from jax.experimental.pallas import tpu as pltpu
from jax.experimental.pallas import tpu_sc as plsc

_NC = 2
_NS = 16
_NW = _NC * _NS
_IDXW = 128


def _sc_gather_mean(x2, table, *, n_patches_total, patch, cp, xrow_base):
    pw = n_patches_total // _NW
    n_chunks = pw // cp
    ct = cp * patch
    n_idx_rows = ct // _IDXW
    assert pw % cp == 0 and ct % _IDXW == 0
    assert n_chunks >= 4 and n_chunks % 2 == 0

    def body(x_hbm, table_hbm, feat_hbm, idx_v0, idx_v1, rows_v0, rows_v1,
             out_v0, out_v1, sem_i0, sem_i1, sem_g0, sem_g1, sem_o0, sem_o1):
        idx_v = (idx_v0, idx_v1)
        rows_v = (rows_v0, rows_v1)
        out_v = (out_v0, out_v1)
        sem_i = (sem_i0, sem_i1)
        sem_g = (sem_g0, sem_g1)
        sem_o = (sem_o0, sem_o1)
        wid = lax.axis_index("s") * _NC + lax.axis_index("c")

        def stage_idx(c, b):
            xrow0 = xrow_base + wid * (n_chunks * n_idx_rows) + c * n_idx_rows
            pltpu.async_copy(x_hbm.at[pl.ds(xrow0, n_idx_rows)], idx_v[b],
                             sem_i[b])

        def fire_gathers(c, b):
            pltpu.make_async_copy(
                x_hbm.at[pl.ds(0, n_idx_rows)], idx_v[b], sem_i[b]).wait()
            for j in range(n_idx_rows):
                pltpu.async_copy(
                    table_hbm.at[idx_v[b].at[j]],
                    rows_v[b].at[pl.ds(j * _IDXW, _IDXW)],
                    sem_g[b])

        def wait_gathers(b):
            for j in range(n_idx_rows):
                pltpu.make_async_copy(
                    table_hbm.at[idx_v[b].at[j]],
                    rows_v[b].at[pl.ds(j * _IDXW, _IDXW)],
                    sem_g[b]).wait()

        def reduce_and_store(c, b, *, wait_out):
            if wait_out:
                pltpu.make_async_copy(
                    out_v[b], feat_hbm.at[pl.ds(0, cp)], sem_o[b]).wait()

            def red(p, carry2):
                r0 = p * patch
                acc = rows_v[b][r0]
                for t in range(1, patch):
                    acc = acc + rows_v[b][r0 + t]
                out_v[b][p] = acc * (1.0 / patch)
                return carry2

            lax.fori_loop(0, cp, red, 0, unroll=4)
            pbase = wid * pw + c * cp
            pltpu.async_copy(out_v[b], feat_hbm.at[pl.ds(pbase, cp)],
                             sem_o[b])

        stage_idx(0, 0)
        stage_idx(1, 1)
        fire_gathers(0, 0)
        fire_gathers(1, 1)
        for c in (0, 1):
            b = c & 1
            wait_gathers(b)
            stage_idx(c + 2, b)
            reduce_and_store(c, b, wait_out=False)
            fire_gathers(c + 2, b)

        def main(i, carry):
            for b in range(2):
                c = 2 + 2 * i + b
                wait_gathers(b)
                stage_idx(c + 2, b)
                reduce_and_store(c, b, wait_out=True)
                fire_gathers(c + 2, b)
            return carry

        if n_chunks > 4:
            lax.fori_loop(0, (n_chunks - 4) // 2, main, 0, unroll=False)

        for c in (n_chunks - 2, n_chunks - 1):
            b = c & 1
            wait_gathers(b)
            reduce_and_store(c, b, wait_out=True)
        for b in range(2):
            pltpu.make_async_copy(
                out_v[b], feat_hbm.at[pl.ds(0, cp)], sem_o[b]).wait()

    k = pl.kernel(
        body,
        out_type=jax.ShapeDtypeStruct((n_patches_total, patch), jnp.float32),
        mesh=plsc.VectorSubcoreMesh(core_axis_name="c", subcore_axis_name="s"),
        scratch_types=[
            pltpu.VMEM((n_idx_rows, _IDXW), jnp.int32),
            pltpu.VMEM((n_idx_rows, _IDXW), jnp.int32),
            pltpu.VMEM((ct, patch), jnp.float32),
            pltpu.VMEM((ct, patch), jnp.float32),
            pltpu.VMEM((cp, patch), jnp.float32),
            pltpu.VMEM((cp, patch), jnp.float32),
            pltpu.SemaphoreType.DMA,
            pltpu.SemaphoreType.DMA,
            pltpu.SemaphoreType.DMA,
            pltpu.SemaphoreType.DMA,
            pltpu.SemaphoreType.DMA,
            pltpu.SemaphoreType.DMA,
        ],
        compiler_params=pltpu.CompilerParams(use_tc_tiling_on_sc=False),
    )
    return k(x2, table)


def _tc_body(buf_ref, feat_ref, w_ref, b_ref, pos_ref, out_ref):
    f = feat_ref[...]
    bb, pp, _ = out_ref.shape
    acc = lax.dot_general(
        f, w_ref[...],
        (((1,), (0,)), ((), ())), preferred_element_type=jnp.float32)
    acc = acc.reshape(bb, pp, -1)
    out_ref[...] = acc + (pos_ref[...] + b_ref[...])[None, :, :]


def _tc_body_noalias(feat_ref, w_ref, b_ref, pos_ref, out_ref):
    _tc_body(None, feat_ref, w_ref, b_ref, pos_ref, out_ref)


def _tc_project_slice(buf, feat_s, W, b, pos2, *, s, b_total, bb):
    npt_s, patch = feat_s.shape
    P_ = pos2.shape[0]
    bs = npt_s // P_
    D_ = W.shape[1]
    nb = bs // bb
    specs = [
        pl.BlockSpec((bb * P_, patch), lambda i: (i, 0)),
        pl.BlockSpec((patch, D_), lambda i: (0, 0)),
        pl.BlockSpec((D_,), lambda i: (0,)),
        pl.BlockSpec((P_, D_), lambda i: (0, 0)),
    ]
    out_spec = pl.BlockSpec((bb, P_, D_), lambda i, s=s: (s * nb + i, 0, 0))
    out_shape = jax.ShapeDtypeStruct((b_total, P_, D_), jnp.float32)
    params = pltpu.CompilerParams(dimension_semantics=("arbitrary",))
    if buf is None:
        return pl.pallas_call(
            _tc_body_noalias, grid=(nb,), in_specs=specs,
            out_specs=out_spec, out_shape=out_shape,
            compiler_params=params,
        )(feat_s, W, b, pos2)
    return pl.pallas_call(
        _tc_body, grid=(nb,),
        in_specs=[pl.BlockSpec((1, 8, 128), lambda i: (0, 0, 0))] + specs,
        out_specs=out_spec, out_shape=out_shape,
        input_output_aliases={0: 0},
        compiler_params=params,
    )(buf, feat_s, W, b, pos2)


def kernel(x, table, W, b, pos_embed):
    B_, S_ = x.shape
    V_, patch = table.shape
    D_ = W.shape[1]
    n_patches = S_ // patch
    pos2 = pos_embed[0, :n_patches, :]

    nsplit = 4
    bs = B_ // nsplit
    npt_s = bs * n_patches
    rows_per_split = npt_s * patch // _IDXW
    x2 = x.reshape(B_ * S_ // _IDXW, _IDXW)
    feats = []
    for s in range(nsplit):
        feats.append(_sc_gather_mean(x2, table, n_patches_total=npt_s,
                                     patch=patch, cp=160,
                                     xrow_base=s * rows_per_split))
    buf = None
    for s in range(nsplit):
        buf = _tc_project_slice(buf, feats[s], W, b, pos2,
                                s=s, b_total=B_, bb=8)
    return buf


# speedup vs baseline: 25.8907x; 1.1115x over previous
import functools

import jax
import jax.numpy as jnp
from jax import lax
from jax.experimental import pallas as pl
from jax.experimental.pallas import tpu as pltpu
from jax.experimental.pallas import tpu_sc as plsc

_NC = 2
_NS = 16
_NW = _NC * _NS
_IDXW = 128


def _sc_gather_mean(x2, table, *, n_patches_total, patch, cp, xrow_base):
    pw = n_patches_total // _NW
    n_chunks = pw // cp
    ct = cp * patch
    n_idx_rows = ct // _IDXW
    assert pw % cp == 0 and ct % _IDXW == 0
    assert n_chunks >= 4 and n_chunks % 2 == 0

    def body(x_hbm, table_hbm, feat_hbm, idx_v0, idx_v1, rows_v0, rows_v1,
             out_v0, out_v1, sem_i0, sem_i1, sem_g0, sem_g1, sem_o0, sem_o1):
        idx_v = (idx_v0, idx_v1)
        rows_v = (rows_v0, rows_v1)
        out_v = (out_v0, out_v1)
        sem_i = (sem_i0, sem_i1)
        sem_g = (sem_g0, sem_g1)
        sem_o = (sem_o0, sem_o1)
        wid = lax.axis_index("s") * _NC + lax.axis_index("c")

        def stage_idx(c, b):
            xrow0 = xrow_base + wid * (n_chunks * n_idx_rows) + c * n_idx_rows
            pltpu.async_copy(x_hbm.at[pl.ds(xrow0, n_idx_rows)], idx_v[b],
                             sem_i[b])

        def fire_gathers(c, b):
            pltpu.make_async_copy(
                x_hbm.at[pl.ds(0, n_idx_rows)], idx_v[b], sem_i[b]).wait()
            for j in range(n_idx_rows):
                pltpu.async_copy(
                    table_hbm.at[idx_v[b].at[j]],
                    rows_v[b].at[pl.ds(j * _IDXW, _IDXW)],
                    sem_g[b])

        def wait_gathers(b):
            for j in range(n_idx_rows):
                pltpu.make_async_copy(
                    table_hbm.at[idx_v[b].at[j]],
                    rows_v[b].at[pl.ds(j * _IDXW, _IDXW)],
                    sem_g[b]).wait()

        def reduce_and_store(c, b, *, wait_out):
            if wait_out:
                pltpu.make_async_copy(
                    out_v[b], feat_hbm.at[pl.ds(0, cp)], sem_o[b]).wait()

            def red(p, carry2):
                r0 = p * patch
                acc = rows_v[b][r0]
                for t in range(1, patch):
                    acc = acc + rows_v[b][r0 + t]
                out_v[b][p] = acc * (1.0 / patch)
                return carry2

            lax.fori_loop(0, cp, red, 0, unroll=4)
            pbase = wid * pw + c * cp
            pltpu.async_copy(out_v[b], feat_hbm.at[pl.ds(pbase, cp)],
                             sem_o[b])

        stage_idx(0, 0)
        stage_idx(1, 1)
        fire_gathers(0, 0)
        fire_gathers(1, 1)
        for c in (0, 1):
            b = c & 1
            wait_gathers(b)
            stage_idx(c + 2, b)
            reduce_and_store(c, b, wait_out=False)
            fire_gathers(c + 2, b)

        def main(i, carry):
            for b in range(2):
                c = 2 + 2 * i + b
                wait_gathers(b)
                stage_idx(c + 2, b)
                reduce_and_store(c, b, wait_out=True)
                fire_gathers(c + 2, b)
            return carry

        if n_chunks > 4:
            lax.fori_loop(0, (n_chunks - 4) // 2, main, 0, unroll=False)

        for c in (n_chunks - 2, n_chunks - 1):
            b = c & 1
            wait_gathers(b)
            reduce_and_store(c, b, wait_out=True)
        for b in range(2):
            pltpu.make_async_copy(
                out_v[b], feat_hbm.at[pl.ds(0, cp)], sem_o[b]).wait()

    k = pl.kernel(
        body,
        out_type=jax.ShapeDtypeStruct((n_patches_total, patch), jnp.float32),
        mesh=plsc.VectorSubcoreMesh(core_axis_name="c", subcore_axis_name="s"),
        scratch_types=[
            pltpu.VMEM((n_idx_rows, _IDXW), jnp.int32),
            pltpu.VMEM((n_idx_rows, _IDXW), jnp.int32),
            pltpu.VMEM((ct, patch), jnp.float32),
            pltpu.VMEM((ct, patch), jnp.float32),
            pltpu.VMEM((cp, patch), jnp.float32),
            pltpu.VMEM((cp, patch), jnp.float32),
            pltpu.SemaphoreType.DMA,
            pltpu.SemaphoreType.DMA,
            pltpu.SemaphoreType.DMA,
            pltpu.SemaphoreType.DMA,
            pltpu.SemaphoreType.DMA,
            pltpu.SemaphoreType.DMA,
        ],
        compiler_params=pltpu.CompilerParams(use_tc_tiling_on_sc=False),
    )
    return k(x2, table)


def _tc_body(buf_ref, feat_ref, w_ref, b_ref, pos_ref, out_ref):
    f = feat_ref[...]
    bb, pp, _ = out_ref.shape
    acc = lax.dot_general(
        f, w_ref[...],
        (((1,), (0,)), ((), ())), preferred_element_type=jnp.float32)
    acc = acc.reshape(bb, pp, -1)
    out_ref[...] = acc + (pos_ref[...] + b_ref[...])[None, :, :]


def _tc_body_noalias(feat_ref, w_ref, b_ref, pos_ref, out_ref):
    _tc_body(None, feat_ref, w_ref, b_ref, pos_ref, out_ref)


def _tc_project_slice(buf, feat_s, W, b, pos2, *, s, b_total, bb):
    npt_s, patch = feat_s.shape
    P_ = pos2.shape[0]
    bs = npt_s // P_
    D_ = W.shape[1]
    nb = bs // bb
    specs = [
        pl.BlockSpec((bb * P_, patch), lambda i: (i, 0)),
        pl.BlockSpec((patch, D_), lambda i: (0, 0)),
        pl.BlockSpec((D_,), lambda i: (0,)),
        pl.BlockSpec((P_, D_), lambda i: (0, 0)),
    ]
    out_spec = pl.BlockSpec((bb, P_, D_), lambda i, s=s: (s * nb + i, 0, 0))
    out_shape = jax.ShapeDtypeStruct((b_total, P_, D_), jnp.float32)
    params = pltpu.CompilerParams(dimension_semantics=("arbitrary",))
    if buf is None:
        return pl.pallas_call(
            _tc_body_noalias, grid=(nb,), in_specs=specs,
            out_specs=out_spec, out_shape=out_shape,
            compiler_params=params,
        )(feat_s, W, b, pos2)
    return pl.pallas_call(
        _tc_body, grid=(nb,),
        in_specs=[pl.BlockSpec((1, 8, 128), lambda i: (0, 0, 0))] + specs,
        out_specs=out_spec, out_shape=out_shape,
        input_output_aliases={0: 0},
        compiler_params=params,
    )(buf, feat_s, W, b, pos2)


def kernel(x, table, W, b, pos_embed):
    B_, S_ = x.shape
    V_, patch = table.shape
    D_ = W.shape[1]
    n_patches = S_ // patch
    pos2 = pos_embed[0, :n_patches, :]

    nsplit = 4
    bs = B_ // nsplit
    npt_s = bs * n_patches
    rows_per_split = npt_s * patch // _IDXW
    x2 = x.reshape(B_ * S_ // _IDXW, _IDXW)
    feats = []
    for s in range(nsplit):
        feats.append(_sc_gather_mean(x2, table, n_patches_total=npt_s,
                                     patch=patch, cp=160,
                                     xrow_base=s * rows_per_split))
    buf = _tc_project_slice(None, feats[0], W, b, pos2,
                            s=0, b_total=B_, bb=8)
    for s in range(1, nsplit):
        feat_s, buf_dep = lax.optimization_barrier((feats[s], buf))
        buf = _tc_project_slice(buf_dep, feat_s, W, b, pos2,
                                s=s, b_total=B_, bb=8)
    return buf


# speedup vs baseline: 31.5879x; 1.2200x over previous
import functools

import jax
import jax.numpy as jnp
from jax import lax
from jax.experimental import pallas as pl
from jax.experimental.pallas import tpu as pltpu
from jax.experimental.pallas import tpu_sc as plsc

_NC = 2
_NS = 16
_NW = _NC * _NS
_IDXW = 128


def _sc_gather_mean(x, table, *, row_base, n_rows, patch):
    S = x.shape[1]
    n_patches = S // patch
    ct = n_patches * patch
    n_seg = ct // _IDXW
    ppr = _IDXW // patch
    orows = n_patches // ppr
    rw = n_rows // _NW
    n_chunks = rw
    assert n_rows % _NW == 0 and ct % _IDXW == 0 and n_patches % ppr == 0
    assert n_chunks >= 4 and n_chunks % 2 == 0

    def body(x_hbm, table_hbm, feat_hbm, idx_v0, idx_v1, rows_v0, rows_v1,
             out_v0, out_v1, sem_i0, sem_i1, sem_g0, sem_g1, sem_o0, sem_o1):
        idx_v = (idx_v0, idx_v1)
        rows_v = (rows_v0, rows_v1)
        out_v = (out_v0, out_v1)
        sem_i = (sem_i0, sem_i1)
        sem_g = (sem_g0, sem_g1)
        sem_o = (sem_o0, sem_o1)
        wid = lax.axis_index("s") * _NC + lax.axis_index("c")

        def stage_idx(c, b):
            xrow = row_base + wid * rw + c
            pltpu.async_copy(x_hbm.at[pl.ds(xrow, 1)], idx_v[b], sem_i[b])

        def fire_gathers(c, b):
            pltpu.make_async_copy(
                x_hbm.at[pl.ds(0, 1)], idx_v[b], sem_i[b]).wait()
            for j in range(n_seg):
                pltpu.async_copy(
                    table_hbm.at[idx_v[b].at[0, pl.ds(j * _IDXW, _IDXW)]],
                    rows_v[b].at[pl.ds(j * _IDXW, _IDXW)],
                    sem_g[b])

        def wait_gathers(b):
            for j in range(n_seg):
                pltpu.make_async_copy(
                    table_hbm.at[idx_v[b].at[0, pl.ds(j * _IDXW, _IDXW)]],
                    rows_v[b].at[pl.ds(j * _IDXW, _IDXW)],
                    sem_g[b]).wait()

        def feat_dst(c):
            return feat_hbm.at[pl.ds(wid * n_patches, n_patches),
                               pl.ds(c * patch, patch)]

        def reduce_and_store(c, b, *, wait_out):
            if wait_out:
                pltpu.make_async_copy(
                    out_v[b], feat_dst(0), sem_o[b]).wait()

            def red(p, carry2):
                r0 = p * patch
                acc = rows_v[b][r0]
                for u in range(1, patch):
                    acc = acc + rows_v[b][r0 + u]
                out_v[b][p] = acc * (1.0 / patch)
                return carry2

            lax.fori_loop(0, n_patches, red, 0, unroll=4)
            pltpu.async_copy(out_v[b], feat_dst(c), sem_o[b])

        stage_idx(0, 0)
        stage_idx(1, 1)
        fire_gathers(0, 0)
        fire_gathers(1, 1)
        for c in (0, 1):
            b = c & 1
            wait_gathers(b)
            stage_idx(c + 2, b)
            reduce_and_store(c, b, wait_out=False)
            fire_gathers(c + 2, b)

        def main(i, carry):
            for b in range(2):
                c = 2 + 2 * i + b
                wait_gathers(b)
                stage_idx(c + 2, b)
                reduce_and_store(c, b, wait_out=True)
                fire_gathers(c + 2, b)
            return carry

        if n_chunks > 4:
            lax.fori_loop(0, (n_chunks - 4) // 2, main, 0, unroll=False)

        for c in (n_chunks - 2, n_chunks - 1):
            b = c & 1
            wait_gathers(b)
            reduce_and_store(c, b, wait_out=True)
        for b in range(2):
            pltpu.make_async_copy(
                out_v[b], feat_dst(0), sem_o[b]).wait()

    k = pl.kernel(
        body,
        out_type=jax.ShapeDtypeStruct((n_rows * S // _IDXW, _IDXW),
                                      jnp.float32),
        mesh=plsc.VectorSubcoreMesh(core_axis_name="c", subcore_axis_name="s"),
        scratch_types=[
            pltpu.VMEM((1, ct), jnp.int32),
            pltpu.VMEM((1, ct), jnp.int32),
            pltpu.VMEM((ct, patch), jnp.float32),
            pltpu.VMEM((ct, patch), jnp.float32),
            pltpu.VMEM((n_patches, patch), jnp.float32),
            pltpu.VMEM((n_patches, patch), jnp.float32),
            pltpu.SemaphoreType.DMA,
            pltpu.SemaphoreType.DMA,
            pltpu.SemaphoreType.DMA,
            pltpu.SemaphoreType.DMA,
            pltpu.SemaphoreType.DMA,
            pltpu.SemaphoreType.DMA,
        ],
        compiler_params=pltpu.CompilerParams(use_tc_tiling_on_sc=False),
    )
    return k(x, table)


def _tc_body(buf_ref, feat_ref, w_ref, b_ref, pos_ref, out_ref):
    f = feat_ref[...]
    bb = out_ref.shape[0]
    k = w_ref.shape[0]
    w = w_ref[...]
    addv = pos_ref[...] + b_ref[...]
    for t in range(bb):
        ft = f[:, t * k:(t + 1) * k]
        acc = lax.dot_general(
            ft, w, (((1,), (0,)), ((), ())),
            preferred_element_type=jnp.float32)
        out_ref[t] = acc + addv


def _tc_body_noalias(feat_ref, w_ref, b_ref, pos_ref, out_ref):
    _tc_body(None, feat_ref, w_ref, b_ref, pos_ref, out_ref)


def _tc_project_slice(buf, feat_s, W, b, pos2, *, s, b_total, bb):
    frows, fw = feat_s.shape
    P_ = pos2.shape[0]
    patch = W.shape[0]
    bs = frows * fw // (P_ * patch)
    D_ = W.shape[1]
    assert bb == fw // patch
    nb = bs // bb
    specs = [
        pl.BlockSpec((P_, fw), lambda i: (i, 0)),
        pl.BlockSpec((patch, D_), lambda i: (0, 0)),
        pl.BlockSpec((D_,), lambda i: (0,)),
        pl.BlockSpec((P_, D_), lambda i: (0, 0)),
    ]
    out_spec = pl.BlockSpec((bb, P_, D_), lambda i, s=s: (s * nb + i, 0, 0))
    out_shape = jax.ShapeDtypeStruct((b_total, P_, D_), jnp.float32)
    params = pltpu.CompilerParams(dimension_semantics=("arbitrary",))
    if buf is None:
        return pl.pallas_call(
            _tc_body_noalias, grid=(nb,), in_specs=specs,
            out_specs=out_spec, out_shape=out_shape,
            compiler_params=params,
        )(feat_s, W, b, pos2)
    return pl.pallas_call(
        _tc_body, grid=(nb,),
        in_specs=[pl.BlockSpec((1, 8, 128), lambda i: (0, 0, 0))] + specs,
        out_specs=out_spec, out_shape=out_shape,
        input_output_aliases={0: 0},
        compiler_params=params,
    )(buf, feat_s, W, b, pos2)


def kernel(x, table, W, b, pos_embed):
    B_, S_ = x.shape
    V_, patch = table.shape
    D_ = W.shape[1]
    n_patches = S_ // patch
    pos2 = pos_embed[0, :n_patches, :]

    nsplit = 4
    bs = B_ // nsplit
    feats = []
    for s in range(nsplit):
        feats.append(_sc_gather_mean(x, table, row_base=s * bs,
                                     n_rows=bs, patch=patch))
    buf = _tc_project_slice(None, feats[0], W, b, pos2,
                            s=0, b_total=B_, bb=8)
    for s in range(1, nsplit):
        feat_s, buf_dep = lax.optimization_barrier((feats[s], buf))
        buf = _tc_project_slice(buf_dep, feat_s, W, b, pos2,
                                s=s, b_total=B_, bb=8)
    return buf
